# baseline (speedup 1.0000x reference)
import functools
import ml_dtypes
import numpy as np
import jax
import jax.numpy as jnp
from jax import lax
from jax.experimental import pallas as pl
from jax.experimental.pallas import tpu as pltpu
from jax.experimental.pallas import tpu_sc as plsc

CODEBOOK = 8192
D_EMB = 256
E_DIM = 2048
DIM = 1024
PACKED = E_DIM // 2
PACKED_D = D_EMB // 2

NW = 32
CHUNK = 128
N_IDX = 65536
PER_W = N_IDX // NW
N_CHUNK = PER_W // CHUNK


@functools.cache
def _make_gather(n_idx):
    mesh = plsc.VectorSubcoreMesh(core_axis_name="c", subcore_axis_name="s")
    per_w = n_idx // NW
    n_chunk = per_w // CHUNK

    @functools.partial(
        pl.kernel,
        out_type=jax.ShapeDtypeStruct((n_idx, PACKED_D), jnp.int32),
        mesh=mesh,
        scratch_types=[
            pltpu.VMEM((n_chunk, CHUNK), jnp.int32),
            pltpu.VMEM((CHUNK, PACKED_D), jnp.int32),
            pltpu.VMEM((CHUNK, PACKED_D), jnp.int32),
            pltpu.SemaphoreType.DMA,
            pltpu.SemaphoreType.DMA,
        ],
    )
    def gather_k(table_hbm, idx_hbm, out_hbm, idx_v, buf0, buf1, sem0, sem1):
        wid = lax.axis_index("s") * 2 + lax.axis_index("c")
        pltpu.sync_copy(idx_hbm.at[pl.ds(wid * n_chunk, n_chunk)], idx_v)
        bufs = (buf0, buf1)
        sems = (sem0, sem1)
        descs = [None, None]
        descs[0] = pltpu.async_copy(table_hbm.at[idx_v.at[0]], bufs[0], sems[0])
        for c in range(n_chunk):
            if c + 1 < n_chunk:
                descs[(c + 1) % 2] = pltpu.async_copy(
                    table_hbm.at[idx_v.at[c + 1]], bufs[(c + 1) % 2],
                    sems[(c + 1) % 2])
            descs[c % 2].wait()
            pltpu.sync_copy(
                bufs[c % 2],
                out_hbm.at[pl.ds(wid * per_w + c * CHUNK, CHUNK)])

    return gather_k


_BM = 512
_SL = 2048


def _rb16(u):
    return (u + jnp.uint32(0x7FFF) + ((u >> 16) & jnp.uint32(1))) >> 16


def _unpack_lo(w):
    return lax.bitcast_convert_type(w << 16, jnp.float32)


def _unpack_hi(w):
    return lax.bitcast_convert_type(w & jnp.int32(-65536), jnp.float32)


def _mm_body(alpha_ref, emb_ref, pe_lo_ref, pe_hi_ref, w_ref, out_ref,
             we_s, wo_s):
    i = pl.program_id(0)
    j = pl.program_id(1)

    @pl.when(jnp.logical_and(i == 0, j == 0))
    def _():
        wraw = w_ref[...]
        we_s[...] = _unpack_lo(wraw).astype(jnp.bfloat16)
        wo_s[...] = _unpack_hi(wraw).astype(jnp.bfloat16)

    raw = emb_ref[0]
    a = alpha_ref[0]
    ze = _unpack_lo(raw) + a * pe_lo_ref[...].astype(jnp.float32)
    zo = _unpack_hi(raw) + a * pe_hi_ref[...].astype(jnp.float32)
    dn = (((1,), (1,)), ((), ()))
    out_ref[0] = (
        lax.dot_general(ze.astype(jnp.bfloat16), we_s[...], dn,
                        preferred_element_type=jnp.float32)
        + lax.dot_general(zo.astype(jnp.bfloat16), wo_s[...], dn,
                          preferred_element_type=jnp.float32))


def _matmul(alpha, emb3, pe_lo, pe_hi, w_packed):
    bs = emb3.shape[0]
    grid = (_SL // _BM, bs)
    return pl.pallas_call(
        _mm_body,
        grid=grid,
        in_specs=[
            pl.BlockSpec(memory_space=pltpu.SMEM),
            pl.BlockSpec((1, _BM, PACKED), lambda i, j: (j, i, 0)),
            pl.BlockSpec((_BM, PACKED), lambda i, j: (i, 0)),
            pl.BlockSpec((_BM, PACKED), lambda i, j: (i, 0)),
            pl.BlockSpec((DIM, PACKED), lambda i, j: (0, 0)),
        ],
        out_specs=pl.BlockSpec((1, _BM, DIM), lambda i, j: (j, i, 0)),
        out_shape=jax.ShapeDtypeStruct((bs, _SL, DIM), jnp.float32),
        scratch_shapes=[
            pltpu.VMEM((DIM, PACKED), jnp.bfloat16),
            pltpu.VMEM((DIM, PACKED), jnp.bfloat16),
        ],
    )(alpha, emb3, pe_lo, pe_hi, w_packed)


@functools.cache
def _pe_tables():
    pos = np.arange(16384, dtype=np.float32)[:, None]
    div = np.exp(np.arange(0, D_EMB, 2, dtype=np.float32)
                 * np.float32(-np.log(10000.0) / D_EMB)).astype(np.float32)
    arg = (pos * div).astype(np.float32).astype(np.float64)
    pe = np.zeros((16384, D_EMB), dtype=np.float32)
    pe[:, 0::2] = np.sin(arg)
    pe[:, 1::2] = np.cos(arg)
    pe3 = pe.reshape(_SL, 8, D_EMB)
    lo = pe3[:, :, :PACKED_D].reshape(_SL, PACKED)
    hi = pe3[:, :, PACKED_D:].reshape(_SL, PACKED)
    return (lo.astype(ml_dtypes.bfloat16), hi.astype(ml_dtypes.bfloat16))


def kernel(x, W_emb, alpha, W_out):
    bs, sl, P = x.shape

    u = lax.bitcast_convert_type(W_emb, jnp.uint32)
    word = _rb16(u[:, :PACKED_D]) | (_rb16(u[:, PACKED_D:]) << 16)
    table = lax.bitcast_convert_type(word, jnp.int32)

    wu = lax.bitcast_convert_type(W_out, jnp.uint32).reshape(DIM, 8, D_EMB)
    wword = (_rb16(wu[:, :, :PACKED_D]) | (_rb16(wu[:, :, PACKED_D:]) << 16))
    w_packed = lax.bitcast_convert_type(wword, jnp.int32).reshape(DIM, PACKED)

    pe_lo, pe_hi = _pe_tables()
    pe_lo = jnp.asarray(pe_lo)
    pe_hi = jnp.asarray(pe_hi)

    half = bs // 2
    n_half = N_IDX // 2
    gather = _make_gather(n_half)
    outs = []
    for h in range(2):
        xh = lax.slice_in_dim(x, h * half, (h + 1) * half, axis=0)
        idx = xh.reshape(n_half // CHUNK, CHUNK)
        emb = gather(table, idx)
        emb3 = emb.reshape(half, sl, PACKED)
        outs.append(_matmul(alpha, emb3, pe_lo, pe_hi, w_packed))
    return jnp.concatenate(outs, axis=0)


# speedup vs baseline: 1.2626x; 1.2626x over previous
import functools
import ml_dtypes
import numpy as np
import jax
import jax.numpy as jnp
from jax import lax
from jax.experimental import pallas as pl
from jax.experimental.pallas import tpu as pltpu
from jax.experimental.pallas import tpu_sc as plsc

CODEBOOK = 8192
D_EMB = 256
E_DIM = 2048
DIM = 1024
PACKED = E_DIM // 2
PACKED_D = D_EMB // 2

NW = 32
CHUNK = 128
N_IDX = 65536
PER_W = N_IDX // NW
N_CHUNK = PER_W // CHUNK


@functools.cache
def _make_gather(n_idx):
    mesh = plsc.VectorSubcoreMesh(core_axis_name="c", subcore_axis_name="s")
    per_w = n_idx // NW
    n_chunk = per_w // CHUNK

    @functools.partial(
        pl.kernel,
        out_type=jax.ShapeDtypeStruct((n_idx, PACKED_D), jnp.int32),
        mesh=mesh,
        scratch_types=[
            pltpu.VMEM((n_chunk, CHUNK), jnp.int32),
            pltpu.VMEM((CHUNK, PACKED_D), jnp.int32),
            pltpu.VMEM((CHUNK, PACKED_D), jnp.int32),
            pltpu.SemaphoreType.DMA,
            pltpu.SemaphoreType.DMA,
        ],
    )
    def gather_k(table_hbm, idx_hbm, out_hbm, idx_v, buf0, buf1, sem0, sem1):
        wid = lax.axis_index("s") * 2 + lax.axis_index("c")
        pltpu.sync_copy(idx_hbm.at[pl.ds(wid * n_chunk, n_chunk)], idx_v)
        bufs = (buf0, buf1)
        sems = (sem0, sem1)
        descs = [None, None]
        descs[0] = pltpu.async_copy(table_hbm.at[idx_v.at[0]], bufs[0], sems[0])
        for c in range(n_chunk):
            if c + 1 < n_chunk:
                descs[(c + 1) % 2] = pltpu.async_copy(
                    table_hbm.at[idx_v.at[c + 1]], bufs[(c + 1) % 2],
                    sems[(c + 1) % 2])
            descs[c % 2].wait()
            pltpu.sync_copy(
                bufs[c % 2],
                out_hbm.at[pl.ds(wid * per_w + c * CHUNK, CHUNK)])

    return gather_k


_BM = 512
_SL = 2048


def _rb16(u):
    return (u + jnp.uint32(0x7FFF) + ((u >> 16) & jnp.uint32(1))) >> 16


def _unpack_lo(w):
    return lax.bitcast_convert_type(w << 16, jnp.float32)


def _unpack_hi(w):
    return lax.bitcast_convert_type(w & jnp.int32(-65536), jnp.float32)


def _mm_body(alpha_ref, emb_ref, pe_lo_ref, pe_hi_ref, w_ref, out_ref,
             we_s, wo_s):
    i = pl.program_id(0)
    j = pl.program_id(1)

    @pl.when(jnp.logical_and(i == 0, j == 0))
    def _():
        w = w_ref[...]
        we_s[...] = jnp.concatenate(
            [w[:, g * D_EMB:g * D_EMB + PACKED_D] for g in range(8)],
            axis=1).astype(jnp.bfloat16)
        wo_s[...] = jnp.concatenate(
            [w[:, g * D_EMB + PACKED_D:(g + 1) * D_EMB] for g in range(8)],
            axis=1).astype(jnp.bfloat16)

    raw = emb_ref[0]
    a = alpha_ref[0]
    ze = _unpack_lo(raw) + a * pe_lo_ref[...].astype(jnp.float32)
    zo = _unpack_hi(raw) + a * pe_hi_ref[...].astype(jnp.float32)
    dn = (((1,), (1,)), ((), ()))
    out_ref[0] = (
        lax.dot_general(ze.astype(jnp.bfloat16), we_s[...], dn,
                        preferred_element_type=jnp.float32)
        + lax.dot_general(zo.astype(jnp.bfloat16), wo_s[...], dn,
                          preferred_element_type=jnp.float32))


def _matmul(alpha, emb3, pe_lo, pe_hi, w_out):
    bs = emb3.shape[0]
    grid = (_SL // _BM, bs)
    return pl.pallas_call(
        _mm_body,
        grid=grid,
        in_specs=[
            pl.BlockSpec(memory_space=pltpu.SMEM),
            pl.BlockSpec((1, _BM, PACKED), lambda i, j: (j, i, 0)),
            pl.BlockSpec((_BM, PACKED), lambda i, j: (i, 0)),
            pl.BlockSpec((_BM, PACKED), lambda i, j: (i, 0)),
            pl.BlockSpec((DIM, E_DIM), lambda i, j: (0, 0)),
        ],
        out_specs=pl.BlockSpec((1, _BM, DIM), lambda i, j: (j, i, 0)),
        out_shape=jax.ShapeDtypeStruct((bs, _SL, DIM), jnp.float32),
        scratch_shapes=[
            pltpu.VMEM((DIM, PACKED), jnp.bfloat16),
            pltpu.VMEM((DIM, PACKED), jnp.bfloat16),
        ],
    )(alpha, emb3, pe_lo, pe_hi, w_out)


@functools.cache
def _pe_tables():
    pos = np.arange(16384, dtype=np.float32)[:, None]
    div = np.exp(np.arange(0, D_EMB, 2, dtype=np.float32)
                 * np.float32(-np.log(10000.0) / D_EMB)).astype(np.float32)
    arg = (pos * div).astype(np.float32).astype(np.float64)
    pe = np.zeros((16384, D_EMB), dtype=np.float32)
    pe[:, 0::2] = np.sin(arg)
    pe[:, 1::2] = np.cos(arg)
    pe3 = pe.reshape(_SL, 8, D_EMB)
    lo = pe3[:, :, :PACKED_D].reshape(_SL, PACKED)
    hi = pe3[:, :, PACKED_D:].reshape(_SL, PACKED)
    return (lo.astype(ml_dtypes.bfloat16), hi.astype(ml_dtypes.bfloat16))


def kernel(x, W_emb, alpha, W_out):
    bs, sl, P = x.shape

    u = lax.bitcast_convert_type(W_emb, jnp.uint32)
    word = _rb16(u[:, :PACKED_D]) | (_rb16(u[:, PACKED_D:]) << 16)
    table = lax.bitcast_convert_type(word, jnp.int32)

    pe_lo, pe_hi = _pe_tables()

    idx = x.reshape(N_IDX // CHUNK, CHUNK)
    emb = _make_gather(N_IDX)(table, idx)
    emb3 = emb.reshape(bs, sl, PACKED)
    out = _matmul(alpha, emb3, jnp.asarray(pe_lo), jnp.asarray(pe_hi), W_out)
    return out.reshape(bs, sl, DIM)
